# Initial kernel scaffold; baseline (speedup 1.0000x reference)
import jax
import jax.numpy as jnp
from jax.experimental import pallas as pl


def kernel(x, edge_index, W1, b1, W2, b2, Wl, bl):
    raise NotImplementedError("write your pallas kernel here")


# speedup vs baseline: 26.5330x; 26.5330x over previous
import functools

import jax
import jax.numpy as jnp
from jax import lax
from jax.experimental import pallas as pl
from jax.experimental.pallas import tpu as pltpu
from jax.experimental.pallas import tpu_sc as plsc

_NC = 2
_NS = 16
_CHUNK = 128


def _make_sc_scatter(n_nodes, n_edges, feat):
    nw = _NC * _NS
    ew = n_edges // nw
    assert ew * nw == n_edges
    n_full = ew // _CHUNK
    rem = ew - n_full * _CHUNK
    rows_per_tile = n_nodes // _NS
    assert rows_per_tile * _NS == n_nodes

    mesh = plsc.VectorSubcoreMesh(core_axis_name="c", subcore_axis_name="s")

    scratch = [
        pltpu.VMEM_SHARED((n_nodes, feat), jnp.float32),
        pltpu.VMEM((_CHUNK,), jnp.int32),
        pltpu.VMEM((_CHUNK,), jnp.int32),
        pltpu.VMEM((_CHUNK, feat), jnp.float32),
        pltpu.SemaphoreType.DMA,
    ]
    if rem:
        scratch += [
            pltpu.VMEM((rem,), jnp.int32),
            pltpu.VMEM((rem,), jnp.int32),
            pltpu.VMEM((rem, feat), jnp.float32),
        ]

    @functools.partial(
        pl.kernel,
        out_type=jax.ShapeDtypeStruct((_NC, n_nodes, feat), jnp.float32),
        mesh=mesh,
        scratch_types=scratch,
        compiler_params=pltpu.CompilerParams(use_tc_tiling_on_sc=False),
    )
    def sc_scatter(src_hbm, row_hbm, col_hbm, zero_hbm, out_hbm,
                   acc, ridx, cidx, rows, sem, *rest):
        cid = lax.axis_index("c")
        sid = lax.axis_index("s")
        wid = sid * _NC + cid

        r0 = sid * rows_per_tile
        pltpu.sync_copy(zero_hbm.at[pl.ds(r0, rows_per_tile)],
                        acc.at[pl.ds(r0, rows_per_tile)])
        plsc.subcore_barrier()

        e0 = wid * ew

        @pl.loop(0, n_full)
        def _edge_chunk(i):
            base = pl.multiple_of(e0 + i * _CHUNK, 8)
            pltpu.sync_copy(row_hbm.at[pl.ds(base, _CHUNK)], ridx)
            pltpu.sync_copy(col_hbm.at[pl.ds(base, _CHUNK)], cidx)
            pltpu.async_copy(src_hbm.at[ridx], rows, sem).wait()
            pltpu.sync_copy(rows, acc.at[cidx], add=True)

        if rem:
            ridx2, cidx2, rows2 = rest
            base = pl.multiple_of(e0 + n_full * _CHUNK, 8)
            pltpu.sync_copy(row_hbm.at[pl.ds(base, rem)], ridx2)
            pltpu.sync_copy(col_hbm.at[pl.ds(base, rem)], cidx2)
            pltpu.async_copy(src_hbm.at[ridx2], rows2, sem).wait()
            pltpu.sync_copy(rows2, acc.at[cidx2], add=True)

        plsc.subcore_barrier()
        pltpu.sync_copy(acc.at[pl.ds(r0, rows_per_tile)],
                        out_hbm.at[cid, pl.ds(r0, rows_per_tile)])

    return sc_scatter


_ROW_BLK = 4000


def _dense1_body(s1_ref, x_ref, w1t_ref, b1_ref, w2t_ref, g_ref):
    s = s1_ref[0] + s1_ref[1]
    a = s[:, :4] + x_ref[...]
    indeg1 = s[:, 4:5] + 1.0
    h1 = jnp.tanh(
        jnp.dot(a, w1t_ref[...], preferred_element_type=jnp.float32)
        + indeg1 * b1_ref[...])
    g = jnp.dot(h1, w2t_ref[...], preferred_element_type=jnp.float32)
    g_ref[...] = jnp.concatenate(
        [g, jnp.zeros((g.shape[0], 5), jnp.float32)], axis=1)


def _dense2_body(s2_ref, g_ref, s1_ref, b2_ref, out_ref):
    indeg1 = s1_ref[0, :, 4:5] + s1_ref[1, :, 4:5] + 1.0
    h2 = jnp.tanh(s2_ref[0] + s2_ref[1] + g_ref[...] + indeg1 * b2_ref[...])
    p0 = jnp.maximum(h2[:, 0:1], h2[:, 1:2])
    p1 = jnp.maximum(jnp.maximum(h2[:, 2:3], h2[:, 3:4]), h2[:, 4:5])
    p2 = jnp.maximum(jnp.maximum(h2[:, 5:6], h2[:, 6:7]), h2[:, 7:8])
    p3 = jnp.maximum(jnp.maximum(h2[:, 8:9], h2[:, 9:10]), h2[:, 10:11])
    out_ref[...] = jnp.concatenate([p0, p1, p2, p3], axis=1)


def _head_body(r_ref, m_ref, bl_ref, out_ref):
    logits = jnp.dot(r_ref[...], m_ref[...],
                     preferred_element_type=jnp.float32) + bl_ref[...]
    mx = jnp.max(logits, axis=1, keepdims=True)
    e = jnp.exp(logits - mx)
    out_ref[...] = e / jnp.sum(e, axis=1, keepdims=True)


def kernel(x, edge_index, W1, b1, W2, b2, Wl, bl):
    n, _ = x.shape
    e = edge_index.shape[1]
    row = edge_index[0]
    col = edge_index[1]

    f1, f2 = 8, 16
    xpad = jnp.concatenate(
        [x, jnp.ones((n, 1), jnp.float32), jnp.zeros((n, 3), jnp.float32)],
        axis=1)
    zeros1 = jnp.zeros((n, f1), jnp.float32)
    zeros2 = jnp.zeros((n, f2), jnp.float32)

    s1 = _make_sc_scatter(n, e, f1)(xpad, row, col, zeros1)

    nb = n // _ROW_BLK
    g = pl.pallas_call(
        _dense1_body,
        grid=(nb,),
        in_specs=[
            pl.BlockSpec((_NC, _ROW_BLK, f1), lambda i: (0, i, 0)),
            pl.BlockSpec((_ROW_BLK, 4), lambda i: (i, 0)),
            pl.BlockSpec((4, 26), lambda i: (0, 0)),
            pl.BlockSpec((1, 26), lambda i: (0, 0)),
            pl.BlockSpec((26, 11), lambda i: (0, 0)),
        ],
        out_specs=pl.BlockSpec((_ROW_BLK, f2), lambda i: (i, 0)),
        out_shape=jax.ShapeDtypeStruct((n, f2), jnp.float32),
    )(s1, x, W1.T, b1.reshape(1, 26), W2.T)

    s2 = _make_sc_scatter(n, e, f2)(g, row, col, zeros2)

    b2pad = jnp.concatenate([b2, jnp.zeros((5,), jnp.float32)]).reshape(1, f2)
    pooled = pl.pallas_call(
        _dense2_body,
        grid=(nb,),
        in_specs=[
            pl.BlockSpec((_NC, _ROW_BLK, f2), lambda i: (0, i, 0)),
            pl.BlockSpec((_ROW_BLK, f2), lambda i: (i, 0)),
            pl.BlockSpec((_NC, _ROW_BLK, f1), lambda i: (0, i, 0)),
            pl.BlockSpec((1, f2), lambda i: (0, 0)),
        ],
        out_specs=pl.BlockSpec((_ROW_BLK, 4), lambda i: (i, 0)),
        out_shape=jax.ShapeDtypeStruct((n, 4), jnp.float32),
    )(s2, g, s1, b2pad)

    num_chunks = (n + 25) // 26
    pad_rows = num_chunks * 26 - n
    r = jnp.pad(pooled, ((0, pad_rows), (0, 0))).reshape(num_chunks, 104)
    m = jnp.tile(Wl.T, (26, 1))
    out = pl.pallas_call(
        _head_body,
        in_specs=[
            pl.BlockSpec((num_chunks, 104), lambda: (0, 0)),
            pl.BlockSpec((104, 2), lambda: (0, 0)),
            pl.BlockSpec((1, 2), lambda: (0, 0)),
        ],
        out_specs=pl.BlockSpec((num_chunks, 2), lambda: (0, 0)),
        out_shape=jax.ShapeDtypeStruct((num_chunks, 2), jnp.float32),
    )(r, m, bl.reshape(1, 2))
    return out


# speedup vs baseline: 95.6713x; 3.6057x over previous
import functools

import jax
import jax.numpy as jnp
from jax import lax
from jax.experimental import pallas as pl
from jax.experimental.pallas import tpu as pltpu
from jax.experimental.pallas import tpu_sc as plsc

_NC = 2
_NS = 16
_CHUNK = 128


_K = 8


def _make_sc_scatter(n_nodes, n_edges, feat):
    nw = _NC * _NS
    ew = n_edges // nw
    assert ew * nw == n_edges
    group = _K * _CHUNK
    n_groups = ew // group
    tail = ew - n_groups * group
    tail_full = tail // _CHUNK
    rem = tail - tail_full * _CHUNK
    rows_per_tile = n_nodes // _NS
    assert rows_per_tile * _NS == n_nodes

    mesh = plsc.VectorSubcoreMesh(core_axis_name="c", subcore_axis_name="s")

    scratch = [
        pltpu.VMEM_SHARED((n_nodes, feat), jnp.float32),
        pltpu.VMEM((_K, 2, _CHUNK), jnp.int32),
        pltpu.VMEM((_K, _CHUNK, feat), jnp.float32),
        pltpu.SemaphoreType.DMA,
        pltpu.SemaphoreType.DMA,
        pltpu.SemaphoreType.DMA,
    ]
    if rem:
        scratch += [
            pltpu.VMEM((2, rem), jnp.int32),
            pltpu.VMEM((rem, feat), jnp.float32),
        ]

    @functools.partial(
        pl.kernel,
        out_type=jax.ShapeDtypeStruct((_NC, n_nodes, feat), jnp.float32),
        mesh=mesh,
        scratch_types=scratch,
        compiler_params=pltpu.CompilerParams(use_tc_tiling_on_sc=False),
    )
    def sc_scatter(src_hbm, ei_hbm, zero_hbm, out_hbm,
                   acc, eidx, rows, isem, gsem, ssem, *rest):
        cid = lax.axis_index("c")
        sid = lax.axis_index("s")
        wid = sid * _NC + cid

        r0 = sid * rows_per_tile
        pltpu.sync_copy(zero_hbm.at[pl.ds(r0, rows_per_tile)],
                        acc.at[pl.ds(r0, rows_per_tile)])
        plsc.subcore_barrier()

        e0 = wid * ew

        @pl.loop(0, n_groups)
        def _edge_group(j):
            base0 = pl.multiple_of(e0 + j * group, 8)
            ds_ = [pltpu.async_copy(
                ei_hbm.at[:, pl.ds(base0 + k * _CHUNK, _CHUNK)],
                eidx.at[k], isem) for k in range(_K)]
            for d in ds_:
                d.wait()
            ds_ = [pltpu.async_copy(
                src_hbm.at[eidx.at[k, 0]], rows.at[k], gsem)
                for k in range(_K)]
            for d in ds_:
                d.wait()
            ds_ = [pltpu.async_copy(
                rows.at[k], acc.at[eidx.at[k, 1]], ssem, add=True)
                for k in range(_K)]
            for d in ds_:
                d.wait()

        tbase = e0 + n_groups * group
        for t in range(tail_full):
            base = pl.multiple_of(tbase + t * _CHUNK, 8)
            pltpu.sync_copy(ei_hbm.at[:, pl.ds(base, _CHUNK)], eidx.at[0])
            pltpu.async_copy(src_hbm.at[eidx.at[0, 0]], rows.at[0], gsem).wait()
            pltpu.sync_copy(rows.at[0], acc.at[eidx.at[0, 1]], add=True)
        if rem:
            eidx_r, rows_r = rest
            base = pl.multiple_of(tbase + tail_full * _CHUNK, 8)
            pltpu.sync_copy(ei_hbm.at[:, pl.ds(base, rem)], eidx_r)
            pltpu.async_copy(src_hbm.at[eidx_r.at[0]], rows_r, gsem).wait()
            pltpu.sync_copy(rows_r, acc.at[eidx_r.at[1]], add=True)

        plsc.subcore_barrier()
        pltpu.sync_copy(acc.at[pl.ds(r0, rows_per_tile)],
                        out_hbm.at[cid, pl.ds(r0, rows_per_tile)])

    return sc_scatter


_ROW_BLK = 4000


def _dense1_body(s1_ref, x_ref, w1t_ref, b1_ref, w2t_ref, g_ref):
    s = s1_ref[0] + s1_ref[1]
    a = s[:, :4] + x_ref[...]
    indeg1 = s[:, 4:5] + 1.0
    h1 = jnp.tanh(
        jnp.dot(a, w1t_ref[...], preferred_element_type=jnp.float32)
        + indeg1 * b1_ref[...])
    g = jnp.dot(h1, w2t_ref[...], preferred_element_type=jnp.float32)
    g_ref[...] = jnp.concatenate(
        [g, jnp.zeros((g.shape[0], 5), jnp.float32)], axis=1)


def _dense2_body(s2_ref, g_ref, s1_ref, b2_ref, out_ref):
    indeg1 = s1_ref[0, :, 4:5] + s1_ref[1, :, 4:5] + 1.0
    h2 = jnp.tanh(s2_ref[0] + s2_ref[1] + g_ref[...] + indeg1 * b2_ref[...])
    p0 = jnp.maximum(h2[:, 0:1], h2[:, 1:2])
    p1 = jnp.maximum(jnp.maximum(h2[:, 2:3], h2[:, 3:4]), h2[:, 4:5])
    p2 = jnp.maximum(jnp.maximum(h2[:, 5:6], h2[:, 6:7]), h2[:, 7:8])
    p3 = jnp.maximum(jnp.maximum(h2[:, 8:9], h2[:, 9:10]), h2[:, 10:11])
    out_ref[...] = jnp.concatenate([p0, p1, p2, p3], axis=1)


def _head_body(r_ref, m_ref, bl_ref, out_ref):
    logits = jnp.dot(r_ref[...], m_ref[...],
                     preferred_element_type=jnp.float32) + bl_ref[...]
    mx = jnp.max(logits, axis=1, keepdims=True)
    e = jnp.exp(logits - mx)
    out_ref[...] = e / jnp.sum(e, axis=1, keepdims=True)


def kernel(x, edge_index, W1, b1, W2, b2, Wl, bl):
    n, _ = x.shape
    e = edge_index.shape[1]

    f1, f2 = 8, 16
    xpad = jnp.concatenate(
        [x, jnp.ones((n, 1), jnp.float32), jnp.zeros((n, 3), jnp.float32)],
        axis=1)
    zeros1 = jnp.zeros((n, f1), jnp.float32)
    zeros2 = jnp.zeros((n, f2), jnp.float32)

    s1 = _make_sc_scatter(n, e, f1)(xpad, edge_index, zeros1)

    nb = n // _ROW_BLK
    g = pl.pallas_call(
        _dense1_body,
        grid=(nb,),
        in_specs=[
            pl.BlockSpec((_NC, _ROW_BLK, f1), lambda i: (0, i, 0)),
            pl.BlockSpec((_ROW_BLK, 4), lambda i: (i, 0)),
            pl.BlockSpec((4, 26), lambda i: (0, 0)),
            pl.BlockSpec((1, 26), lambda i: (0, 0)),
            pl.BlockSpec((26, 11), lambda i: (0, 0)),
        ],
        out_specs=pl.BlockSpec((_ROW_BLK, f2), lambda i: (i, 0)),
        out_shape=jax.ShapeDtypeStruct((n, f2), jnp.float32),
    )(s1, x, W1.T, b1.reshape(1, 26), W2.T)

    s2 = _make_sc_scatter(n, e, f2)(g, edge_index, zeros2)

    b2pad = jnp.concatenate([b2, jnp.zeros((5,), jnp.float32)]).reshape(1, f2)
    pooled = pl.pallas_call(
        _dense2_body,
        grid=(nb,),
        in_specs=[
            pl.BlockSpec((_NC, _ROW_BLK, f2), lambda i: (0, i, 0)),
            pl.BlockSpec((_ROW_BLK, f2), lambda i: (i, 0)),
            pl.BlockSpec((_NC, _ROW_BLK, f1), lambda i: (0, i, 0)),
            pl.BlockSpec((1, f2), lambda i: (0, 0)),
        ],
        out_specs=pl.BlockSpec((_ROW_BLK, 4), lambda i: (i, 0)),
        out_shape=jax.ShapeDtypeStruct((n, 4), jnp.float32),
    )(s2, g, s1, b2pad)

    num_chunks = (n + 25) // 26
    pad_rows = num_chunks * 26 - n
    r = jnp.pad(pooled, ((0, pad_rows), (0, 0))).reshape(num_chunks, 104)
    m = jnp.tile(Wl.T, (26, 1))
    out = pl.pallas_call(
        _head_body,
        in_specs=[
            pl.BlockSpec((num_chunks, 104), lambda: (0, 0)),
            pl.BlockSpec((104, 2), lambda: (0, 0)),
            pl.BlockSpec((1, 2), lambda: (0, 0)),
        ],
        out_specs=pl.BlockSpec((num_chunks, 2), lambda: (0, 0)),
        out_shape=jax.ShapeDtypeStruct((num_chunks, 2), jnp.float32),
    )(r, m, bl.reshape(1, 2))
    return out


# speedup vs baseline: 107.2932x; 1.1215x over previous
import functools

import jax
import jax.numpy as jnp
from jax import lax
from jax.experimental import pallas as pl
from jax.experimental.pallas import tpu as pltpu
from jax.experimental.pallas import tpu_sc as plsc

_NC = 2
_NS = 16
_CHUNK = 128


_K = 16


def _make_sc_scatter(n_nodes, n_edges, feat):
    nw = _NC * _NS
    ew = n_edges // nw
    assert ew * nw == n_edges
    group = _K * _CHUNK
    n_groups = ew // group
    tail = ew - n_groups * group
    tail_full = tail // _CHUNK
    rem = tail - tail_full * _CHUNK
    assert tail_full <= _K
    rows_per_tile = n_nodes // _NS
    assert rows_per_tile * _NS == n_nodes

    mesh = plsc.VectorSubcoreMesh(core_axis_name="c", subcore_axis_name="s")

    scratch = [
        pltpu.VMEM_SHARED((n_nodes, feat), jnp.float32),
        pltpu.VMEM((_K, 2, _CHUNK), jnp.int32),
        pltpu.VMEM((_K, _CHUNK, feat), jnp.float32),
        pltpu.SemaphoreType.DMA,
        pltpu.SemaphoreType.DMA,
        pltpu.SemaphoreType.DMA,
    ]
    if rem:
        scratch += [
            pltpu.VMEM((2, rem), jnp.int32),
            pltpu.VMEM((rem, feat), jnp.float32),
        ]

    @functools.partial(
        pl.kernel,
        out_type=jax.ShapeDtypeStruct((_NC, n_nodes, feat), jnp.float32),
        mesh=mesh,
        scratch_types=scratch,
        compiler_params=pltpu.CompilerParams(use_tc_tiling_on_sc=False),
    )
    def sc_scatter(src_hbm, ei_hbm, zero_hbm, out_hbm,
                   acc, eidx, rows, isem, gsem, ssem, *rest):
        cid = lax.axis_index("c")
        sid = lax.axis_index("s")
        wid = sid * _NC + cid

        r0 = sid * rows_per_tile
        pltpu.sync_copy(zero_hbm.at[pl.ds(r0, rows_per_tile)],
                        acc.at[pl.ds(r0, rows_per_tile)])
        plsc.subcore_barrier()

        e0 = wid * ew

        def fire_drain_group(base0, nk):
            ds_ = [pltpu.async_copy(
                ei_hbm.at[:, pl.ds(base0 + k * _CHUNK, _CHUNK)],
                eidx.at[k], isem) for k in range(nk)]
            for d in ds_:
                d.wait()
            ds_ = [pltpu.async_copy(
                src_hbm.at[eidx.at[k, 0]], rows.at[k], gsem)
                for k in range(nk)]
            for d in ds_:
                d.wait()
            ds_ = [pltpu.async_copy(
                rows.at[k], acc.at[eidx.at[k, 1]], ssem, add=True)
                for k in range(nk)]
            for d in ds_:
                d.wait()

        @pl.loop(0, n_groups)
        def _edge_group(j):
            fire_drain_group(pl.multiple_of(e0 + j * group, 8), _K)

        tbase = e0 + n_groups * group
        if tail_full:
            fire_drain_group(pl.multiple_of(tbase, 8), tail_full)
        if rem:
            eidx_r, rows_r = rest
            base = pl.multiple_of(tbase + tail_full * _CHUNK, 8)
            pltpu.sync_copy(ei_hbm.at[:, pl.ds(base, rem)], eidx_r)
            pltpu.async_copy(src_hbm.at[eidx_r.at[0]], rows_r, gsem).wait()
            pltpu.sync_copy(rows_r, acc.at[eidx_r.at[1]], add=True)

        plsc.subcore_barrier()
        pltpu.sync_copy(acc.at[pl.ds(r0, rows_per_tile)],
                        out_hbm.at[cid, pl.ds(r0, rows_per_tile)])

    return sc_scatter


_ROW_BLK = 4000


def _dense1_body(s1_ref, x_ref, w1t_ref, b1_ref, w2t_ref, g_ref):
    s = s1_ref[0] + s1_ref[1]
    a = s[:, :4] + x_ref[...]
    indeg1 = s[:, 4:5] + 1.0
    h1 = jnp.tanh(
        jnp.dot(a, w1t_ref[...], preferred_element_type=jnp.float32)
        + indeg1 * b1_ref[...])
    g = jnp.dot(h1, w2t_ref[...], preferred_element_type=jnp.float32)
    g_ref[...] = jnp.concatenate(
        [g, jnp.zeros((g.shape[0], 1), jnp.float32)], axis=1)


def _dense2_body(s2_ref, g_ref, s1_ref, b2_ref, out_ref):
    indeg1 = s1_ref[0, :, 4:5] + s1_ref[1, :, 4:5] + 1.0
    h2 = jnp.tanh(s2_ref[0] + s2_ref[1] + g_ref[...] + indeg1 * b2_ref[...])
    p0 = jnp.maximum(h2[:, 0:1], h2[:, 1:2])
    p1 = jnp.maximum(jnp.maximum(h2[:, 2:3], h2[:, 3:4]), h2[:, 4:5])
    p2 = jnp.maximum(jnp.maximum(h2[:, 5:6], h2[:, 6:7]), h2[:, 7:8])
    p3 = jnp.maximum(jnp.maximum(h2[:, 8:9], h2[:, 9:10]), h2[:, 10:11])
    out_ref[...] = jnp.concatenate([p0, p1, p2, p3], axis=1)


def _head_body(r_ref, m_ref, bl_ref, out_ref):
    logits = jnp.dot(r_ref[...], m_ref[...],
                     preferred_element_type=jnp.float32) + bl_ref[...]
    mx = jnp.max(logits, axis=1, keepdims=True)
    e = jnp.exp(logits - mx)
    out_ref[...] = e / jnp.sum(e, axis=1, keepdims=True)


def kernel(x, edge_index, W1, b1, W2, b2, Wl, bl):
    n, _ = x.shape
    e = edge_index.shape[1]

    f1, f2 = 8, 12
    xpad = jnp.concatenate(
        [x, jnp.ones((n, 1), jnp.float32), jnp.zeros((n, 3), jnp.float32)],
        axis=1)
    zeros1 = jnp.zeros((n, f1), jnp.float32)
    zeros2 = jnp.zeros((n, f2), jnp.float32)

    s1 = _make_sc_scatter(n, e, f1)(xpad, edge_index, zeros1)

    nb = n // _ROW_BLK
    g = pl.pallas_call(
        _dense1_body,
        grid=(nb,),
        in_specs=[
            pl.BlockSpec((_NC, _ROW_BLK, f1), lambda i: (0, i, 0)),
            pl.BlockSpec((_ROW_BLK, 4), lambda i: (i, 0)),
            pl.BlockSpec((4, 26), lambda i: (0, 0)),
            pl.BlockSpec((1, 26), lambda i: (0, 0)),
            pl.BlockSpec((26, 11), lambda i: (0, 0)),
        ],
        out_specs=pl.BlockSpec((_ROW_BLK, f2), lambda i: (i, 0)),
        out_shape=jax.ShapeDtypeStruct((n, f2), jnp.float32),
    )(s1, x, W1.T, b1.reshape(1, 26), W2.T)

    s2 = _make_sc_scatter(n, e, f2)(g, edge_index, zeros2)

    b2pad = jnp.concatenate([b2, jnp.zeros((1,), jnp.float32)]).reshape(1, f2)
    pooled = pl.pallas_call(
        _dense2_body,
        grid=(nb,),
        in_specs=[
            pl.BlockSpec((_NC, _ROW_BLK, f2), lambda i: (0, i, 0)),
            pl.BlockSpec((_ROW_BLK, f2), lambda i: (i, 0)),
            pl.BlockSpec((_NC, _ROW_BLK, f1), lambda i: (0, i, 0)),
            pl.BlockSpec((1, f2), lambda i: (0, 0)),
        ],
        out_specs=pl.BlockSpec((_ROW_BLK, 4), lambda i: (i, 0)),
        out_shape=jax.ShapeDtypeStruct((n, 4), jnp.float32),
    )(s2, g, s1, b2pad)

    num_chunks = (n + 25) // 26
    pad_rows = num_chunks * 26 - n
    r = jnp.pad(pooled, ((0, pad_rows), (0, 0))).reshape(num_chunks, 104)
    m = jnp.tile(Wl.T, (26, 1))
    out = pl.pallas_call(
        _head_body,
        in_specs=[
            pl.BlockSpec((num_chunks, 104), lambda: (0, 0)),
            pl.BlockSpec((104, 2), lambda: (0, 0)),
            pl.BlockSpec((1, 2), lambda: (0, 0)),
        ],
        out_specs=pl.BlockSpec((num_chunks, 2), lambda: (0, 0)),
        out_shape=jax.ShapeDtypeStruct((num_chunks, 2), jnp.float32),
    )(r, m, bl.reshape(1, 2))
    return out


# speedup vs baseline: 108.2192x; 1.0086x over previous
import functools

import jax
import jax.numpy as jnp
from jax import lax
from jax.experimental import pallas as pl
from jax.experimental.pallas import tpu as pltpu
from jax.experimental.pallas import tpu_sc as plsc

_NC = 2
_NS = 16
_CHUNK = 128


def _make_sc_scatter(n_nodes, n_edges, feat, n_streams):
    _K = n_streams
    nw = _NC * _NS
    ew = n_edges // nw
    assert ew * nw == n_edges
    group = _K * _CHUNK
    n_groups = ew // group
    tail = ew - n_groups * group
    tail_full = tail // _CHUNK
    rem = tail - tail_full * _CHUNK
    assert tail_full <= _K
    rows_per_tile = n_nodes // _NS
    assert rows_per_tile * _NS == n_nodes

    mesh = plsc.VectorSubcoreMesh(core_axis_name="c", subcore_axis_name="s")

    scratch = [
        pltpu.VMEM_SHARED((n_nodes, feat), jnp.float32),
        pltpu.VMEM((_K, 2, _CHUNK), jnp.int32),
        pltpu.VMEM((_K, _CHUNK, feat), jnp.float32),
        pltpu.SemaphoreType.DMA,
        pltpu.SemaphoreType.DMA,
        pltpu.SemaphoreType.DMA,
    ]
    if rem:
        scratch += [
            pltpu.VMEM((2, rem), jnp.int32),
            pltpu.VMEM((rem, feat), jnp.float32),
        ]

    @functools.partial(
        pl.kernel,
        out_type=jax.ShapeDtypeStruct((_NC, n_nodes, feat), jnp.float32),
        mesh=mesh,
        scratch_types=scratch,
        compiler_params=pltpu.CompilerParams(use_tc_tiling_on_sc=False),
    )
    def sc_scatter(src_hbm, ei_hbm, zero_hbm, out_hbm,
                   acc, eidx, rows, isem, gsem, ssem, *rest):
        cid = lax.axis_index("c")
        sid = lax.axis_index("s")
        wid = sid * _NC + cid

        r0 = sid * rows_per_tile
        pltpu.sync_copy(zero_hbm.at[pl.ds(r0, rows_per_tile)],
                        acc.at[pl.ds(r0, rows_per_tile)])
        plsc.subcore_barrier()

        e0 = wid * ew

        def fire_drain_group(base0, nk):
            ds_ = [pltpu.async_copy(
                ei_hbm.at[:, pl.ds(base0 + k * _CHUNK, _CHUNK)],
                eidx.at[k], isem) for k in range(nk)]
            for d in ds_:
                d.wait()
            ds_ = [pltpu.async_copy(
                src_hbm.at[eidx.at[k, 0]], rows.at[k], gsem)
                for k in range(nk)]
            for d in ds_:
                d.wait()
            ds_ = [pltpu.async_copy(
                rows.at[k], acc.at[eidx.at[k, 1]], ssem, add=True)
                for k in range(nk)]
            for d in ds_:
                d.wait()

        @pl.loop(0, n_groups)
        def _edge_group(j):
            fire_drain_group(pl.multiple_of(e0 + j * group, 8), _K)

        tbase = e0 + n_groups * group
        if tail_full:
            fire_drain_group(pl.multiple_of(tbase, 8), tail_full)
        if rem:
            eidx_r, rows_r = rest
            base = pl.multiple_of(tbase + tail_full * _CHUNK, 8)
            pltpu.sync_copy(ei_hbm.at[:, pl.ds(base, rem)], eidx_r)
            pltpu.async_copy(src_hbm.at[eidx_r.at[0]], rows_r, gsem).wait()
            pltpu.sync_copy(rows_r, acc.at[eidx_r.at[1]], add=True)

        plsc.subcore_barrier()
        pltpu.sync_copy(acc.at[pl.ds(r0, rows_per_tile)],
                        out_hbm.at[cid, pl.ds(r0, rows_per_tile)])

    return sc_scatter


_ROW_BLK = 4000


def _dense1_body(s1_ref, x_ref, w1t_ref, b1_ref, w2t_ref, g_ref):
    s = s1_ref[0] + s1_ref[1]
    a = s[:, :4] + x_ref[...]
    indeg1 = s[:, 4:5] + 1.0
    h1 = jnp.tanh(
        jnp.dot(a, w1t_ref[...], preferred_element_type=jnp.float32)
        + indeg1 * b1_ref[...])
    g = jnp.dot(h1, w2t_ref[...], preferred_element_type=jnp.float32)
    g_ref[...] = jnp.concatenate(
        [g, jnp.zeros((g.shape[0], 5), jnp.float32)], axis=1)


def _dense2_body(s2_ref, g_ref, s1_ref, b2_ref, out_ref):
    indeg1 = s1_ref[0, :, 4:5] + s1_ref[1, :, 4:5] + 1.0
    h2 = jnp.tanh(s2_ref[0] + s2_ref[1] + g_ref[...] + indeg1 * b2_ref[...])
    p0 = jnp.maximum(h2[:, 0:1], h2[:, 1:2])
    p1 = jnp.maximum(jnp.maximum(h2[:, 2:3], h2[:, 3:4]), h2[:, 4:5])
    p2 = jnp.maximum(jnp.maximum(h2[:, 5:6], h2[:, 6:7]), h2[:, 7:8])
    p3 = jnp.maximum(jnp.maximum(h2[:, 8:9], h2[:, 9:10]), h2[:, 10:11])
    out_ref[...] = jnp.concatenate([p0, p1, p2, p3], axis=1)


def _head_body(r_ref, m_ref, bl_ref, out_ref):
    logits = jnp.dot(r_ref[...], m_ref[...],
                     preferred_element_type=jnp.float32) + bl_ref[...]
    mx = jnp.max(logits, axis=1, keepdims=True)
    e = jnp.exp(logits - mx)
    out_ref[...] = e / jnp.sum(e, axis=1, keepdims=True)


def kernel(x, edge_index, W1, b1, W2, b2, Wl, bl):
    n, _ = x.shape
    e = edge_index.shape[1]

    f1, f2 = 8, 16
    xpad = jnp.concatenate(
        [x, jnp.ones((n, 1), jnp.float32), jnp.zeros((n, 3), jnp.float32)],
        axis=1)
    zeros1 = jnp.zeros((n, f1), jnp.float32)
    zeros2 = jnp.zeros((n, f2), jnp.float32)

    s1 = _make_sc_scatter(n, e, f1, 16)(xpad, edge_index, zeros1)

    nb = n // _ROW_BLK
    g = pl.pallas_call(
        _dense1_body,
        grid=(nb,),
        in_specs=[
            pl.BlockSpec((_NC, _ROW_BLK, f1), lambda i: (0, i, 0)),
            pl.BlockSpec((_ROW_BLK, 4), lambda i: (i, 0)),
            pl.BlockSpec((4, 26), lambda i: (0, 0)),
            pl.BlockSpec((1, 26), lambda i: (0, 0)),
            pl.BlockSpec((26, 11), lambda i: (0, 0)),
        ],
        out_specs=pl.BlockSpec((_ROW_BLK, f2), lambda i: (i, 0)),
        out_shape=jax.ShapeDtypeStruct((n, f2), jnp.float32),
    )(s1, x, W1.T, b1.reshape(1, 26), W2.T)

    s2 = _make_sc_scatter(n, e, f2, 12)(g, edge_index, zeros2)

    b2pad = jnp.concatenate([b2, jnp.zeros((5,), jnp.float32)]).reshape(1, f2)
    pooled = pl.pallas_call(
        _dense2_body,
        grid=(nb,),
        in_specs=[
            pl.BlockSpec((_NC, _ROW_BLK, f2), lambda i: (0, i, 0)),
            pl.BlockSpec((_ROW_BLK, f2), lambda i: (i, 0)),
            pl.BlockSpec((_NC, _ROW_BLK, f1), lambda i: (0, i, 0)),
            pl.BlockSpec((1, f2), lambda i: (0, 0)),
        ],
        out_specs=pl.BlockSpec((_ROW_BLK, 4), lambda i: (i, 0)),
        out_shape=jax.ShapeDtypeStruct((n, 4), jnp.float32),
    )(s2, g, s1, b2pad)

    num_chunks = (n + 25) // 26
    pad_rows = num_chunks * 26 - n
    r = jnp.pad(pooled, ((0, pad_rows), (0, 0))).reshape(num_chunks, 104)
    m = jnp.tile(Wl.T, (26, 1))
    out = pl.pallas_call(
        _head_body,
        in_specs=[
            pl.BlockSpec((num_chunks, 104), lambda: (0, 0)),
            pl.BlockSpec((104, 2), lambda: (0, 0)),
            pl.BlockSpec((1, 2), lambda: (0, 0)),
        ],
        out_specs=pl.BlockSpec((num_chunks, 2), lambda: (0, 0)),
        out_shape=jax.ShapeDtypeStruct((num_chunks, 2), jnp.float32),
    )(r, m, bl.reshape(1, 2))
    return out


# speedup vs baseline: 124.3134x; 1.1487x over previous
import functools

import jax
import jax.numpy as jnp
from jax import lax
from jax.experimental import pallas as pl
from jax.experimental.pallas import tpu as pltpu
from jax.experimental.pallas import tpu_sc as plsc

_NC = 2
_NS = 16
_CHUNK = 128


def _make_sc_scatter(n_nodes, n_edges, feat, n_streams):
    _K = n_streams
    nw = _NC * _NS
    ew = n_edges // nw
    assert ew * nw == n_edges
    group = _K * _CHUNK
    n_groups = ew // group
    tail = ew - n_groups * group
    tail_full = tail // _CHUNK
    rem = tail - tail_full * _CHUNK
    assert tail_full <= _K
    rows_per_tile = n_nodes // _NS
    assert rows_per_tile * _NS == n_nodes

    mesh = plsc.VectorSubcoreMesh(core_axis_name="c", subcore_axis_name="s")

    scratch = [
        pltpu.VMEM_SHARED((n_nodes, feat), jnp.float32),
        pltpu.VMEM((_K, 2, _CHUNK), jnp.int32),
        pltpu.VMEM((_K, _CHUNK, feat), jnp.float32),
        pltpu.SemaphoreType.DMA((_K,)),
    ]
    if rem:
        scratch += [
            pltpu.VMEM((2, rem), jnp.int32),
            pltpu.VMEM((rem, feat), jnp.float32),
        ]

    @functools.partial(
        pl.kernel,
        out_type=jax.ShapeDtypeStruct((_NC, n_nodes, feat), jnp.float32),
        mesh=mesh,
        scratch_types=scratch,
        compiler_params=pltpu.CompilerParams(use_tc_tiling_on_sc=False),
    )
    def sc_scatter(src_hbm, ei_hbm, zero_hbm, out_hbm,
                   acc, eidx, rows, sem, *rest):
        cid = lax.axis_index("c")
        sid = lax.axis_index("s")
        wid = sid * _NC + cid

        r0 = sid * rows_per_tile
        pltpu.sync_copy(zero_hbm.at[pl.ds(r0, rows_per_tile)],
                        acc.at[pl.ds(r0, rows_per_tile)])
        plsc.subcore_barrier()

        e0 = wid * ew

        def fire_drain_group(base0, nk):
            ids = [pltpu.async_copy(
                ei_hbm.at[:, pl.ds(base0 + k * _CHUNK, _CHUNK)],
                eidx.at[k], sem.at[k]) for k in range(nk)]
            gs = []
            for k in range(nk):
                ids[k].wait()
                gs.append(pltpu.async_copy(
                    src_hbm.at[eidx.at[k, 0]], rows.at[k], sem.at[k]))
            ss = []
            for k in range(nk):
                gs[k].wait()
                ss.append(pltpu.async_copy(
                    rows.at[k], acc.at[eidx.at[k, 1]], sem.at[k], add=True))
            for k in range(nk):
                ss[k].wait()

        @pl.loop(0, n_groups)
        def _edge_group(j):
            fire_drain_group(pl.multiple_of(e0 + j * group, 8), _K)

        tbase = e0 + n_groups * group
        if tail_full:
            fire_drain_group(pl.multiple_of(tbase, 8), tail_full)
        if rem:
            eidx_r, rows_r = rest
            base = pl.multiple_of(tbase + tail_full * _CHUNK, 8)
            pltpu.sync_copy(ei_hbm.at[:, pl.ds(base, rem)], eidx_r)
            pltpu.async_copy(src_hbm.at[eidx_r.at[0]], rows_r, sem.at[0]).wait()
            pltpu.sync_copy(rows_r, acc.at[eidx_r.at[1]], add=True)

        plsc.subcore_barrier()
        pltpu.sync_copy(acc.at[pl.ds(r0, rows_per_tile)],
                        out_hbm.at[cid, pl.ds(r0, rows_per_tile)])

    return sc_scatter


_ROW_BLK = 4000


def _dense1_body(s1_ref, x_ref, w1t_ref, b1_ref, w2t_ref, g_ref):
    s = s1_ref[0] + s1_ref[1]
    a = s[:, :4] + x_ref[...]
    indeg1 = s[:, 4:5] + 1.0
    h1 = jnp.tanh(
        jnp.dot(a, w1t_ref[...], preferred_element_type=jnp.float32)
        + indeg1 * b1_ref[...])
    g = jnp.dot(h1, w2t_ref[...], preferred_element_type=jnp.float32)
    g_ref[...] = jnp.concatenate(
        [g, jnp.zeros((g.shape[0], 5), jnp.float32)], axis=1)


def _dense2_body(s2_ref, g_ref, s1_ref, b2_ref, out_ref):
    indeg1 = s1_ref[0, :, 4:5] + s1_ref[1, :, 4:5] + 1.0
    h2 = jnp.tanh(s2_ref[0] + s2_ref[1] + g_ref[...] + indeg1 * b2_ref[...])
    p0 = jnp.maximum(h2[:, 0:1], h2[:, 1:2])
    p1 = jnp.maximum(jnp.maximum(h2[:, 2:3], h2[:, 3:4]), h2[:, 4:5])
    p2 = jnp.maximum(jnp.maximum(h2[:, 5:6], h2[:, 6:7]), h2[:, 7:8])
    p3 = jnp.maximum(jnp.maximum(h2[:, 8:9], h2[:, 9:10]), h2[:, 10:11])
    out_ref[...] = jnp.concatenate([p0, p1, p2, p3], axis=1)


def _head_body(r_ref, m_ref, bl_ref, out_ref):
    logits = jnp.dot(r_ref[...], m_ref[...],
                     preferred_element_type=jnp.float32) + bl_ref[...]
    mx = jnp.max(logits, axis=1, keepdims=True)
    e = jnp.exp(logits - mx)
    out_ref[...] = e / jnp.sum(e, axis=1, keepdims=True)


def kernel(x, edge_index, W1, b1, W2, b2, Wl, bl):
    n, _ = x.shape
    e = edge_index.shape[1]

    f1, f2 = 8, 16
    xpad = jnp.concatenate(
        [x, jnp.ones((n, 1), jnp.float32), jnp.zeros((n, 3), jnp.float32)],
        axis=1)
    zeros1 = jnp.zeros((n, f1), jnp.float32)
    zeros2 = jnp.zeros((n, f2), jnp.float32)

    s1 = _make_sc_scatter(n, e, f1, 16)(xpad, edge_index, zeros1)

    nb = n // _ROW_BLK
    g = pl.pallas_call(
        _dense1_body,
        grid=(nb,),
        in_specs=[
            pl.BlockSpec((_NC, _ROW_BLK, f1), lambda i: (0, i, 0)),
            pl.BlockSpec((_ROW_BLK, 4), lambda i: (i, 0)),
            pl.BlockSpec((4, 26), lambda i: (0, 0)),
            pl.BlockSpec((1, 26), lambda i: (0, 0)),
            pl.BlockSpec((26, 11), lambda i: (0, 0)),
        ],
        out_specs=pl.BlockSpec((_ROW_BLK, f2), lambda i: (i, 0)),
        out_shape=jax.ShapeDtypeStruct((n, f2), jnp.float32),
    )(s1, x, W1.T, b1.reshape(1, 26), W2.T)

    s2 = _make_sc_scatter(n, e, f2, 12)(g, edge_index, zeros2)

    b2pad = jnp.concatenate([b2, jnp.zeros((5,), jnp.float32)]).reshape(1, f2)
    pooled = pl.pallas_call(
        _dense2_body,
        grid=(nb,),
        in_specs=[
            pl.BlockSpec((_NC, _ROW_BLK, f2), lambda i: (0, i, 0)),
            pl.BlockSpec((_ROW_BLK, f2), lambda i: (i, 0)),
            pl.BlockSpec((_NC, _ROW_BLK, f1), lambda i: (0, i, 0)),
            pl.BlockSpec((1, f2), lambda i: (0, 0)),
        ],
        out_specs=pl.BlockSpec((_ROW_BLK, 4), lambda i: (i, 0)),
        out_shape=jax.ShapeDtypeStruct((n, 4), jnp.float32),
    )(s2, g, s1, b2pad)

    num_chunks = (n + 25) // 26
    pad_rows = num_chunks * 26 - n
    r = jnp.pad(pooled, ((0, pad_rows), (0, 0))).reshape(num_chunks, 104)
    m = jnp.tile(Wl.T, (26, 1))
    out = pl.pallas_call(
        _head_body,
        in_specs=[
            pl.BlockSpec((num_chunks, 104), lambda: (0, 0)),
            pl.BlockSpec((104, 2), lambda: (0, 0)),
            pl.BlockSpec((1, 2), lambda: (0, 0)),
        ],
        out_specs=pl.BlockSpec((num_chunks, 2), lambda: (0, 0)),
        out_shape=jax.ShapeDtypeStruct((num_chunks, 2), jnp.float32),
    )(r, m, bl.reshape(1, 2))
    return out
